# Initial kernel scaffold; baseline (speedup 1.0000x reference)
import jax
import jax.numpy as jnp
from jax.experimental import pallas as pl


def kernel(x, edge_index, W, b):
    raise NotImplementedError("write your pallas kernel here")


# speedup vs baseline: 24.6307x; 24.6307x over previous
import functools

import jax
import jax.numpy as jnp
from jax import lax
from jax.experimental import pallas as pl
from jax.experimental.pallas import tpu as pltpu
from jax.experimental.pallas import tpu_sc as plsc

N = 10000
D = 128
DH = 64
E = 320000
CH = 128
NCHUNK = 2528
EP = NCHUNK * CH
PAD_ROWS = 112
NR = N + PAD_ROWS
CHUNKS_A = NCHUNK // 32
CHUNKS_C = NCHUNK // 16
ROWS_T = NR // 16

_mesh = plsc.VectorSubcoreMesh(core_axis_name="c", subcore_axis_name="s")


@functools.partial(
    pl.kernel,
    mesh=_mesh,
    out_type=jax.ShapeDtypeStruct((2, NR, 8), jnp.float32),
    scratch_types=[
        pltpu.VMEM((CHUNKS_A, CH), jnp.int32),
        pltpu.VMEM((CH, 8), jnp.float32),
        pltpu.VMEM_SHARED((NR, 8), jnp.float32),
    ],
    compiler_params=pltpu.CompilerParams(use_tc_tiling_on_sc=False),
)
def _deg_kernel(dst_hbm, ones_hbm, zeros_hbm, deg_out, idx_v, ones_v, deg_sh):
    c = lax.axis_index("c")
    s = lax.axis_index("s")
    w = c * 16 + s
    r0 = s * ROWS_T
    pltpu.sync_copy(zeros_hbm.at[pl.ds(r0, ROWS_T)], deg_sh.at[pl.ds(r0, ROWS_T)])
    pltpu.sync_copy(dst_hbm.at[w], idx_v)
    pltpu.sync_copy(ones_hbm, ones_v)
    plsc.subcore_barrier()

    def body(j, carry):
        pltpu.sync_copy(ones_v, deg_sh.at[idx_v.at[j]], add=True)
        return carry

    lax.fori_loop(0, CHUNKS_A, body, 0)
    plsc.subcore_barrier()
    pltpu.sync_copy(deg_sh.at[pl.ds(r0, ROWS_T)], deg_out.at[c, pl.ds(r0, ROWS_T)])


@functools.partial(
    pl.kernel,
    mesh=_mesh,
    out_type=jax.ShapeDtypeStruct((2, NR, DH), jnp.float32),
    scratch_types=[
        pltpu.VMEM((CHUNKS_C, CH), jnp.int32),
        pltpu.VMEM((CHUNKS_C, CH), jnp.int32),
        pltpu.VMEM((CH, DH), jnp.float32),
        pltpu.VMEM_SHARED((NR, DH), jnp.float32),
        pltpu.SemaphoreType.DMA,
    ],
    compiler_params=pltpu.CompilerParams(use_tc_tiling_on_sc=False),
)
def _spmm_kernel(y2_hbm, src_lo_hbm, src_hi_hbm, dst_hbm, out_hbm,
                 src_v, dst_v, rows_v, acc_sh, sem):
    c = lax.axis_index("c")
    s = lax.axis_index("s")
    r0 = s * ROWS_T
    pltpu.sync_copy(y2_hbm.at[pl.ds(c * NR + r0, ROWS_T)],
                    acc_sh.at[pl.ds(r0, ROWS_T)])

    @pl.when(c == 0)
    def _():
        pltpu.sync_copy(src_lo_hbm.at[s], src_v)

    @pl.when(c == 1)
    def _():
        pltpu.sync_copy(src_hi_hbm.at[s], src_v)

    pltpu.sync_copy(dst_hbm.at[s], dst_v)
    plsc.subcore_barrier()

    def body(j, carry):
        pltpu.async_copy(y2_hbm.at[src_v.at[j]], rows_v, sem).wait()
        pltpu.sync_copy(rows_v, acc_sh.at[dst_v.at[j]], add=True)
        return carry

    lax.fori_loop(0, CHUNKS_C, body, 0)
    plsc.subcore_barrier()
    pltpu.sync_copy(acc_sh.at[pl.ds(r0, ROWS_T)], out_hbm.at[c, pl.ds(r0, ROWS_T)])


def _prep_body(x_ref, w_ref, degp_ref, y2_ref, dinv_ref):
    deg = degp_ref[0, :, 0:1] + degp_ref[1, :, 0:1] + 1.0
    dinv = lax.rsqrt(deg)
    xw = jnp.dot(x_ref[...], w_ref[...], preferred_element_type=jnp.float32)
    y = xw * dinv
    y2_ref[0] = y[:, :DH]
    y2_ref[1] = y[:, DH:]
    dinv_ref[...] = dinv


def _final_body(s2_ref, dinv_ref, b_ref, out_ref):
    y = jnp.concatenate([s2_ref[0], s2_ref[1]], axis=1)
    out_ref[...] = y * dinv_ref[...] + b_ref[...]


_BN = 1000

_prep = pl.pallas_call(
    _prep_body,
    grid=(N // _BN,),
    in_specs=[
        pl.BlockSpec((_BN, D), lambda i: (i, 0)),
        pl.BlockSpec((D, D), lambda i: (0, 0)),
        pl.BlockSpec((2, _BN, 8), lambda i: (0, i, 0)),
    ],
    out_specs=[
        pl.BlockSpec((2, _BN, DH), lambda i: (0, i, 0)),
        pl.BlockSpec((_BN, 1), lambda i: (i, 0)),
    ],
    out_shape=[
        jax.ShapeDtypeStruct((2, NR, DH), jnp.float32),
        jax.ShapeDtypeStruct((N, 1), jnp.float32),
    ],
)

_final = pl.pallas_call(
    _final_body,
    grid=(N // _BN,),
    in_specs=[
        pl.BlockSpec((2, _BN, DH), lambda i: (0, i, 0)),
        pl.BlockSpec((_BN, 1), lambda i: (i, 0)),
        pl.BlockSpec((1, D), lambda i: (0, 0)),
    ],
    out_specs=pl.BlockSpec((_BN, D), lambda i: (i, 0)),
    out_shape=jax.ShapeDtypeStruct((N, D), jnp.float32),
)


def kernel(x, edge_index, W, b):
    src = edge_index[0]
    dst = edge_index[1]
    pad = N + (lax.iota(jnp.int32, EP - E) % PAD_ROWS)
    srcp = jnp.concatenate([src, pad])
    dstp = jnp.concatenate([dst, pad])
    ones8 = jnp.ones((CH, 8), jnp.float32)
    zeros8 = jnp.zeros((NR, 8), jnp.float32)
    degp = _deg_kernel(dstp.reshape(32, CHUNKS_A, CH), ones8, zeros8)
    y2, dinv = _prep(x, W, degp)
    s2 = _spmm_kernel(y2.reshape(2 * NR, DH),
                      srcp.reshape(16, CHUNKS_C, CH),
                      (srcp + NR).reshape(16, CHUNKS_C, CH),
                      dstp.reshape(16, CHUNKS_C, CH))
    out = _final(s2, dinv, b.reshape(1, D))
    return out


# speedup vs baseline: 39.2273x; 1.5926x over previous
import functools

import jax
import jax.numpy as jnp
from jax import lax
from jax.experimental import pallas as pl
from jax.experimental.pallas import tpu as pltpu
from jax.experimental.pallas import tpu_sc as plsc

N = 10000
D = 128
DH = 64
E = 320000
CH = 128
NCHUNK = 2560
EP = NCHUNK * CH
PAD_ROWS = 240
NR = N + PAD_ROWS
CHUNKS_A = NCHUNK // 32
CHUNKS_C = NCHUNK // 16
ROWS_T = NR // 16
NB = 2
NGH = CHUNKS_C // (2 * NB)

_mesh = plsc.VectorSubcoreMesh(core_axis_name="c", subcore_axis_name="s")
_untiled = pltpu.CompilerParams(use_tc_tiling_on_sc=False)


@functools.partial(
    pl.kernel,
    mesh=_mesh,
    out_type=jax.ShapeDtypeStruct((2, NR, 1), jnp.float32),
    scratch_types=[
        pltpu.VMEM((CHUNKS_A, CH), jnp.int32),
        pltpu.VMEM((CH, 1), jnp.float32),
        pltpu.VMEM_SHARED((NR, 1), jnp.float32),
        pltpu.SemaphoreType.DMA,
    ],
    compiler_params=_untiled,
)
def _deg_kernel(dst_hbm, ones_hbm, zeros_hbm, deg_out, idx_v, ones_v, deg_sh, sem):
    c = lax.axis_index("c")
    s = lax.axis_index("s")
    w = c * 16 + s
    r0 = s * ROWS_T
    pltpu.sync_copy(zeros_hbm.at[pl.ds(r0, ROWS_T)], deg_sh.at[pl.ds(r0, ROWS_T)])
    pltpu.sync_copy(dst_hbm.at[w], idx_v)
    pltpu.sync_copy(ones_hbm, ones_v)
    plsc.subcore_barrier()

    def body(g, carry):
        hs = [pltpu.async_copy(ones_v, deg_sh.at[idx_v.at[g * 8 + b]], sem,
                               add=True)
              for b in range(8)]
        for h in hs:
            h.wait()
        return carry

    lax.fori_loop(0, CHUNKS_A // 8, body, 0)
    plsc.subcore_barrier()
    pltpu.sync_copy(deg_sh.at[pl.ds(r0, ROWS_T)], deg_out.at[c, pl.ds(r0, ROWS_T)])


@functools.partial(
    pl.kernel,
    mesh=_mesh,
    out_type=jax.ShapeDtypeStruct((2, NR, DH), jnp.float32),
    scratch_types=[
        pltpu.VMEM((CHUNKS_C, CH), jnp.int32),
        pltpu.VMEM((CHUNKS_C, CH), jnp.int32),
        pltpu.VMEM((2 * NB, CH, DH), jnp.float32),
        pltpu.VMEM_SHARED((NR, DH), jnp.float32),
        pltpu.SemaphoreType.DMA,
        pltpu.SemaphoreType.DMA,
        pltpu.SemaphoreType.DMA,
        pltpu.SemaphoreType.DMA,
        pltpu.SemaphoreType.DMA,
    ],
    compiler_params=_untiled,
)
def _spmm_kernel(y2_hbm, src_lo_hbm, src_hi_hbm, dst_hbm, out_hbm,
                 src_v, dst_v, rows_v, acc_sh, gsa, gsb, ssa, ssb, sem):
    c = lax.axis_index("c")
    s = lax.axis_index("s")
    r0 = s * ROWS_T

    h0 = pltpu.async_copy(y2_hbm.at[pl.ds(c * NR + r0, ROWS_T)],
                          acc_sh.at[pl.ds(r0, ROWS_T)], sem)
    h2 = pltpu.async_copy(dst_hbm.at[s], dst_v, sem)

    @pl.when(c == 0)
    def _():
        pltpu.async_copy(src_lo_hbm.at[s], src_v, sem).wait()

    @pl.when(c == 1)
    def _():
        pltpu.async_copy(src_hi_hbm.at[s], src_v, sem).wait()

    h0.wait()
    h2.wait()

    def fire_gather(j, b, gs):
        pltpu.async_copy(y2_hbm.at[src_v.at[j]], rows_v.at[b], gs)

    def drain_gather(b, gs):
        pltpu.make_async_copy(y2_hbm.at[src_v.at[0]], rows_v.at[b], gs).wait()

    def fire_scatter(j, b, ss):
        pltpu.async_copy(rows_v.at[b], acc_sh.at[dst_v.at[j]], ss, add=True)

    def drain_scatter(b, ss):
        pltpu.make_async_copy(rows_v.at[b], acc_sh.at[dst_v.at[0]], ss).wait()

    for b in range(NB):
        fire_gather(b, b, gsa)
    plsc.subcore_barrier()

    def body(k, carry):
        ja = 2 * k * NB
        jb = ja + NB
        for b in range(NB):
            fire_gather(jb + b, NB + b, gsb)
        for b in range(NB):
            drain_gather(b, gsa)
        for b in range(NB):
            fire_scatter(ja + b, b, ssa)
        for b in range(NB):
            drain_scatter(b, ssa)

        @pl.when(k < NGH - 1)
        def _():
            for b in range(NB):
                fire_gather(jb + NB + b, b, gsa)

        for b in range(NB):
            drain_gather(NB + b, gsb)
        for b in range(NB):
            fire_scatter(jb + b, NB + b, ssb)
        for b in range(NB):
            drain_scatter(NB + b, ssb)
        return carry

    lax.fori_loop(0, NGH, body, 0)
    plsc.subcore_barrier()
    pltpu.sync_copy(acc_sh.at[pl.ds(r0, ROWS_T)], out_hbm.at[c, pl.ds(r0, ROWS_T)])


def _prep_body(x_ref, w_ref, degp_ref, y2_ref, dinv_ref):
    deg = degp_ref[0] + degp_ref[1] + 1.0
    dinv = lax.rsqrt(deg)
    xw = jnp.dot(x_ref[...], w_ref[...], preferred_element_type=jnp.float32)
    y = xw * dinv
    y2_ref[0] = y[:, :DH]
    y2_ref[1] = y[:, DH:]
    dinv_ref[...] = dinv


def _final_body(s2_ref, dinv_ref, b_ref, out_ref):
    y = jnp.concatenate([s2_ref[0], s2_ref[1]], axis=1)
    out_ref[...] = y * dinv_ref[...] + b_ref[...]


_BN = 1000

_prep = pl.pallas_call(
    _prep_body,
    grid=(N // _BN,),
    in_specs=[
        pl.BlockSpec((_BN, D), lambda i: (i, 0)),
        pl.BlockSpec((D, D), lambda i: (0, 0)),
        pl.BlockSpec((2, _BN, 1), lambda i: (0, i, 0)),
    ],
    out_specs=[
        pl.BlockSpec((2, _BN, DH), lambda i: (0, i, 0)),
        pl.BlockSpec((_BN, 1), lambda i: (i, 0)),
    ],
    out_shape=[
        jax.ShapeDtypeStruct((2, NR, DH), jnp.float32),
        jax.ShapeDtypeStruct((N, 1), jnp.float32),
    ],
)

_final = pl.pallas_call(
    _final_body,
    grid=(N // _BN,),
    in_specs=[
        pl.BlockSpec((2, _BN, DH), lambda i: (0, i, 0)),
        pl.BlockSpec((_BN, 1), lambda i: (i, 0)),
        pl.BlockSpec((1, D), lambda i: (0, 0)),
    ],
    out_specs=pl.BlockSpec((_BN, D), lambda i: (i, 0)),
    out_shape=jax.ShapeDtypeStruct((N, D), jnp.float32),
)


def kernel(x, edge_index, W, b):
    src = edge_index[0]
    dst = edge_index[1]
    pad = N + (lax.iota(jnp.int32, EP - E) % PAD_ROWS)
    srcp = jnp.concatenate([src, pad])
    dstp = jnp.concatenate([dst, pad])
    ones1 = jnp.ones((CH, 1), jnp.float32)
    zeros1 = jnp.zeros((NR, 1), jnp.float32)
    degp = _deg_kernel(dstp.reshape(32, CHUNKS_A, CH), ones1, zeros1)
    y2, dinv = _prep(x, W, degp)
    s2 = _spmm_kernel(y2.reshape(2 * NR, DH),
                      srcp.reshape(16, CHUNKS_C, CH),
                      (srcp + NR).reshape(16, CHUNKS_C, CH),
                      dstp.reshape(16, CHUNKS_C, CH))
    out = _final(s2, dinv, b.reshape(1, D))
    return out


# speedup vs baseline: 39.6039x; 1.0096x over previous
import functools

import jax
import jax.numpy as jnp
from jax import lax
from jax.experimental import pallas as pl
from jax.experimental.pallas import tpu as pltpu
from jax.experimental.pallas import tpu_sc as plsc

N = 10000
D = 128
DH = 64
E = 320000
CH = 128
NCHUNK = 2560
EP = NCHUNK * CH
PAD_ROWS = 240
NR = N + PAD_ROWS
CHUNKS_A = NCHUNK // 32
CHUNKS_C = NCHUNK // 16
ROWS_T = NR // 16
NB = 2
NGH = CHUNKS_C // (2 * NB)

_mesh = plsc.VectorSubcoreMesh(core_axis_name="c", subcore_axis_name="s")
_untiled = pltpu.CompilerParams(use_tc_tiling_on_sc=False)


@functools.partial(
    pl.kernel,
    mesh=_mesh,
    out_type=jax.ShapeDtypeStruct((2, NR, 1), jnp.float32),
    scratch_types=[
        pltpu.VMEM((CHUNKS_A, CH), jnp.int32),
        pltpu.VMEM((CH, 1), jnp.float32),
        pltpu.VMEM_SHARED((NR, 1), jnp.float32),
        pltpu.SemaphoreType.DMA,
    ],
    compiler_params=_untiled,
)
def _deg_kernel(dst_hbm, ones_hbm, zeros_hbm, deg_out, idx_v, ones_v, deg_sh, sem):
    c = lax.axis_index("c")
    s = lax.axis_index("s")
    w = c * 16 + s
    r0 = s * ROWS_T
    pltpu.sync_copy(zeros_hbm.at[pl.ds(r0, ROWS_T)], deg_sh.at[pl.ds(r0, ROWS_T)])
    pltpu.sync_copy(dst_hbm.at[pl.ds(w * CHUNKS_A, CHUNKS_A)], idx_v)
    pltpu.sync_copy(ones_hbm, ones_v)
    plsc.subcore_barrier()

    def body(g, carry):
        hs = [pltpu.async_copy(ones_v, deg_sh.at[idx_v.at[g * 8 + b]], sem,
                               add=True)
              for b in range(8)]
        for h in hs:
            h.wait()
        return carry

    lax.fori_loop(0, CHUNKS_A // 8, body, 0)
    plsc.subcore_barrier()
    pltpu.sync_copy(deg_sh.at[pl.ds(r0, ROWS_T)], deg_out.at[c, pl.ds(r0, ROWS_T)])


@functools.partial(
    pl.kernel,
    mesh=_mesh,
    out_type=jax.ShapeDtypeStruct((2, NR, DH), jnp.float32),
    scratch_types=[
        pltpu.VMEM((CHUNKS_C, CH), jnp.int32),
        pltpu.VMEM((CHUNKS_C, CH), jnp.int32),
        pltpu.VMEM((2 * NB, CH, DH), jnp.float32),
        pltpu.VMEM_SHARED((NR, DH), jnp.float32),
        pltpu.SemaphoreType.DMA,
        pltpu.SemaphoreType.DMA,
        pltpu.SemaphoreType.DMA,
        pltpu.SemaphoreType.DMA,
        pltpu.SemaphoreType.DMA,
    ],
    compiler_params=_untiled,
)
def _spmm_kernel(y_lo_hbm, y_hi_hbm, src_hbm, dst_hbm, out_hbm,
                 src_v, dst_v, rows_v, acc_sh, gsa, gsb, ssa, ssb, sem):
    c = lax.axis_index("c")
    s = lax.axis_index("s")
    r0 = s * ROWS_T

    h2 = pltpu.async_copy(dst_hbm.at[pl.ds(s * CHUNKS_C, CHUNKS_C)], dst_v, sem)
    h3 = pltpu.async_copy(src_hbm.at[pl.ds(s * CHUNKS_C, CHUNKS_C)], src_v, sem)

    def run(y_hbm):
        pltpu.async_copy(y_hbm.at[pl.ds(r0, ROWS_T)],
                         acc_sh.at[pl.ds(r0, ROWS_T)], sem).wait()
        h2.wait()
        h3.wait()

        def fire_gather(j, b, gs):
            pltpu.async_copy(y_hbm.at[src_v.at[j]], rows_v.at[b], gs)

        def drain_gather(b, gs):
            pltpu.make_async_copy(y_hbm.at[src_v.at[0]], rows_v.at[b], gs).wait()

        def fire_scatter(j, b, ss):
            pltpu.async_copy(rows_v.at[b], acc_sh.at[dst_v.at[j]], ss, add=True)

        def drain_scatter(b, ss):
            pltpu.make_async_copy(rows_v.at[b], acc_sh.at[dst_v.at[0]], ss).wait()

        for b in range(NB):
            fire_gather(b, b, gsa)
        plsc.subcore_barrier()

        def body(k, carry):
            ja = 2 * k * NB
            jb = ja + NB
            for b in range(NB):
                fire_gather(jb + b, NB + b, gsb)
            for b in range(NB):
                drain_gather(b, gsa)
            for b in range(NB):
                fire_scatter(ja + b, b, ssa)
            for b in range(NB):
                drain_scatter(b, ssa)

            @pl.when(k < NGH - 1)
            def _():
                for b in range(NB):
                    fire_gather(jb + NB + b, b, gsa)

            for b in range(NB):
                drain_gather(NB + b, gsb)
            for b in range(NB):
                fire_scatter(jb + b, NB + b, ssb)
            for b in range(NB):
                drain_scatter(NB + b, ssb)
            return carry

        lax.fori_loop(0, NGH, body, 0)
        plsc.subcore_barrier()
        pltpu.sync_copy(acc_sh.at[pl.ds(r0, ROWS_T)],
                        out_hbm.at[c, pl.ds(r0, ROWS_T)])

    @pl.when(c == 0)
    def _():
        run(y_lo_hbm)

    @pl.when(c == 1)
    def _():
        run(y_hi_hbm)


def _prep_body(x_ref, w_ref, degp_ref, ylo_ref, yhi_ref, dinv_ref):
    deg = degp_ref[0] + degp_ref[1] + 1.0
    dinv = lax.rsqrt(deg)
    xw = jnp.dot(x_ref[...], w_ref[...], preferred_element_type=jnp.float32)
    y = xw * dinv
    ylo_ref[...] = y[:, :DH]
    yhi_ref[...] = y[:, DH:]
    dinv_ref[...] = dinv


def _final_body(s2_ref, dinv_ref, b_ref, out_ref):
    y = jnp.concatenate([s2_ref[0], s2_ref[1]], axis=1)
    out_ref[...] = y * dinv_ref[...] + b_ref[...]


_BN = 1000

_prep = pl.pallas_call(
    _prep_body,
    grid=(N // _BN,),
    in_specs=[
        pl.BlockSpec((_BN, D), lambda i: (i, 0)),
        pl.BlockSpec((D, D), lambda i: (0, 0)),
        pl.BlockSpec((2, _BN, 1), lambda i: (0, i, 0)),
    ],
    out_specs=[
        pl.BlockSpec((_BN, DH), lambda i: (i, 0)),
        pl.BlockSpec((_BN, DH), lambda i: (i, 0)),
        pl.BlockSpec((_BN, 1), lambda i: (i, 0)),
    ],
    out_shape=[
        jax.ShapeDtypeStruct((NR, DH), jnp.float32),
        jax.ShapeDtypeStruct((NR, DH), jnp.float32),
        jax.ShapeDtypeStruct((N, 1), jnp.float32),
    ],
)

_final = pl.pallas_call(
    _final_body,
    grid=(N // _BN,),
    in_specs=[
        pl.BlockSpec((2, _BN, DH), lambda i: (0, i, 0)),
        pl.BlockSpec((_BN, 1), lambda i: (i, 0)),
        pl.BlockSpec((1, D), lambda i: (0, 0)),
    ],
    out_specs=pl.BlockSpec((_BN, D), lambda i: (i, 0)),
    out_shape=jax.ShapeDtypeStruct((N, D), jnp.float32),
)


def kernel(x, edge_index, W, b):
    src = edge_index[0]
    dst = edge_index[1]
    pad = N + (lax.iota(jnp.int32, EP - E) % PAD_ROWS)
    srcp = jnp.concatenate([src, pad]).reshape(NCHUNK, CH)
    dstp = jnp.concatenate([dst, pad]).reshape(NCHUNK, CH)
    ones1 = jnp.ones((CH, 1), jnp.float32)
    zeros1 = jnp.zeros((NR, 1), jnp.float32)
    degp = _deg_kernel(dstp, ones1, zeros1)
    y_lo, y_hi, dinv = _prep(x, W, degp)
    s2 = _spmm_kernel(y_lo, y_hi, srcp, dstp)
    out = _final(s2, dinv, b.reshape(1, D))
    return out
